# baseline (speedup 1.0000x reference)
import jax
import jax.numpy as jnp
from jax.experimental import pallas as pl
from jax.experimental.pallas import tpu as pltpu

MU = 0.7
SIGMA = 0.2
BE = 256


def _fused(xs_ref, avail_ref, g_ref, w0_ref, w1_ref, w2_ref, bs_ref,
           wc_ref, bc_ref, out_ref, xw_s, idx_s):
    i = pl.program_id(0)
    nb = pl.num_programs(0)

    @pl.when(i == 0)
    def _init():
        xw_s[...] = jnp.exp(-0.5 * ((xs_ref[...] - MU) / SIGMA) ** 2)
        avail = avail_ref[...]
        logsel = jnp.log(avail / jnp.sum(avail, axis=-1, keepdims=True))
        s0 = g_ref[0] + logsel[:, 0:1]
        s1 = g_ref[1] + logsel[:, 1:2]
        s2 = g_ref[2] + logsel[:, 2:3]
        idx_s[...] = jnp.where(
            s0 >= s1,
            jnp.where(s0 >= s2, 0, 2),
            jnp.where(s1 >= s2, 1, 2),
        ).astype(jnp.int32)

    sl = pl.ds(i * BE, BE)
    dn = (((1,), (1,)), ((), ()))
    hp = jax.lax.Precision.HIGHEST
    d0 = jax.lax.dot_general(xw_s[0], w0_ref[...], dn, precision=hp,
                             preferred_element_type=jnp.float32)
    d1 = jax.lax.dot_general(xw_s[1], w1_ref[...], dn, precision=hp,
                             preferred_element_type=jnp.float32)
    d2 = jax.lax.dot_general(xw_s[2], w2_ref[...], dn, precision=hp,
                             preferred_element_type=jnp.float32)
    d0 = jnp.maximum(d0 + bs_ref[0, sl][None, :], 0.0)
    d1 = jnp.maximum(d1 + bs_ref[1, sl][None, :], 0.0)
    d2 = jnp.maximum(d2 + bs_ref[2, sl][None, :], 0.0)
    idx = idx_s[:, sl]
    emb = jnp.where(idx == 0, d0, jnp.where(idx == 1, d1, d2))
    part = jax.lax.dot_general(emb, wc_ref[...], dn, precision=hp,
                               preferred_element_type=jnp.float32)

    @pl.when(i == 0)
    def _first():
        out_ref[...] = part

    @pl.when(i > 0)
    def _acc():
        out_ref[...] += part

    @pl.when(i == nb - 1)
    def _final():
        l = out_ref[...] + bc_ref[...]
        l = l - jnp.max(l, axis=-1, keepdims=True)
        p = jnp.exp(l)
        out_ref[...] = p / jnp.sum(p, axis=-1, keepdims=True)


def kernel(face, audio, text, availabilities, W0, b0, W1, b1, W2, b2, Wc, bc):
    B, D = face.shape
    E = W0.shape[0]
    C = Wc.shape[0]
    M = availabilities.shape[1]
    xs = jnp.stack([face, audio, text])
    bs = jnp.stack([b0, b1, b2])
    g = jax.random.gumbel(jax.random.key(42), (E, B, M), jnp.float32)
    gT = g.transpose(2, 1, 0)
    bc2 = bc.reshape(1, C)
    nb = E // BE

    return pl.pallas_call(
        _fused,
        grid=(nb,),
        in_specs=[
            pl.BlockSpec((M, B, D), lambda i: (0, 0, 0)),
            pl.BlockSpec((B, M), lambda i: (0, 0)),
            pl.BlockSpec((M, B, E), lambda i: (0, 0, 0)),
            pl.BlockSpec((BE, D), lambda i: (i, 0)),
            pl.BlockSpec((BE, D), lambda i: (i, 0)),
            pl.BlockSpec((BE, D), lambda i: (i, 0)),
            pl.BlockSpec((M, E), lambda i: (0, 0)),
            pl.BlockSpec((C, BE), lambda i: (0, i)),
            pl.BlockSpec((1, C), lambda i: (0, 0)),
        ],
        out_specs=pl.BlockSpec((B, C), lambda i: (0, 0)),
        out_shape=jax.ShapeDtypeStruct((B, C), jnp.float32),
        scratch_shapes=[
            pltpu.VMEM((M, B, D), jnp.float32),
            pltpu.VMEM((B, E), jnp.int32),
        ],
        compiler_params=pltpu.CompilerParams(
            dimension_semantics=("arbitrary",),
        ),
    )(xs, availabilities, gT, W0, W1, W2, bs, Wc, bc2)


# speedup vs baseline: 1.9610x; 1.9610x over previous
import jax
import jax.numpy as jnp
from jax.experimental import pallas as pl
from jax.experimental.pallas import tpu as pltpu

MU = 0.7
SIGMA = 0.2
BE = 256


def _fused(xs_ref, avail_ref, g_ref, w0_ref, w1_ref, w2_ref, bs_ref,
           wc_ref, bc_ref, out_ref, xw_s, idx_s, acc_s):
    i = pl.program_id(0)
    nb = pl.num_programs(0)

    @pl.when(i == 0)
    def _init():
        xw_s[...] = jnp.exp(-0.5 * ((xs_ref[...] - MU) / SIGMA) ** 2)
        avail = avail_ref[...]
        logsel = jnp.log(avail / jnp.sum(avail, axis=-1, keepdims=True))
        s0 = g_ref[0] + logsel[:, 0][None, :]
        s1 = g_ref[1] + logsel[:, 1][None, :]
        s2 = g_ref[2] + logsel[:, 2][None, :]
        idx_s[...] = jnp.where(
            s0 >= s1,
            jnp.where(s0 >= s2, 0, 2),
            jnp.where(s1 >= s2, 1, 2),
        ).astype(jnp.int32)

    sl = pl.ds(i * BE, BE)
    dn = (((1,), (1,)), ((), ()))
    d0 = jax.lax.dot_general(w0_ref[...], xw_s[0], dn,
                             preferred_element_type=jnp.float32)
    d1 = jax.lax.dot_general(w1_ref[...], xw_s[1], dn,
                             preferred_element_type=jnp.float32)
    d2 = jax.lax.dot_general(w2_ref[...], xw_s[2], dn,
                             preferred_element_type=jnp.float32)
    d0 = jnp.maximum(d0 + bs_ref[0, sl][:, None], 0.0)
    d1 = jnp.maximum(d1 + bs_ref[1, sl][:, None], 0.0)
    d2 = jnp.maximum(d2 + bs_ref[2, sl][:, None], 0.0)
    idx = idx_s[sl, :]
    emb = jnp.where(idx == 0, d0, jnp.where(idx == 1, d1, d2))
    part = jax.lax.dot_general(wc_ref[...], emb, (((1,), (0,)), ((), ())),
                               preferred_element_type=jnp.float32)

    @pl.when(i == 0)
    def _first():
        acc_s[...] = part

    @pl.when(i > 0)
    def _acc():
        acc_s[...] += part

    @pl.when(i == nb - 1)
    def _final():
        l = acc_s[...] + bc_ref[...]
        l = l - jnp.max(l, axis=0, keepdims=True)
        p = jnp.exp(l)
        out_ref[...] = p / jnp.sum(p, axis=0, keepdims=True)


def kernel(face, audio, text, availabilities, W0, b0, W1, b1, W2, b2, Wc, bc):
    B, D = face.shape
    E = W0.shape[0]
    C = Wc.shape[0]
    M = availabilities.shape[1]
    xs = jnp.stack([face, audio, text])
    bs = jnp.stack([b0, b1, b2])
    g = jax.random.gumbel(jax.random.key(42), (E, B, M), jnp.float32)
    gT = g.transpose(2, 0, 1)
    bc2 = bc.reshape(C, 1)
    nb = E // BE

    out = pl.pallas_call(
        _fused,
        grid=(nb,),
        in_specs=[
            pl.BlockSpec((M, B, D), lambda i: (0, 0, 0)),
            pl.BlockSpec((B, M), lambda i: (0, 0)),
            pl.BlockSpec((M, E, B), lambda i: (0, 0, 0)),
            pl.BlockSpec((BE, D), lambda i: (i, 0)),
            pl.BlockSpec((BE, D), lambda i: (i, 0)),
            pl.BlockSpec((BE, D), lambda i: (i, 0)),
            pl.BlockSpec((M, E), lambda i: (0, 0)),
            pl.BlockSpec((C, BE), lambda i: (0, i)),
            pl.BlockSpec((C, 1), lambda i: (0, 0)),
        ],
        out_specs=pl.BlockSpec((C, B), lambda i: (0, 0)),
        out_shape=jax.ShapeDtypeStruct((C, B), jnp.float32),
        scratch_shapes=[
            pltpu.VMEM((M, B, D), jnp.float32),
            pltpu.VMEM((E, B), jnp.int32),
            pltpu.VMEM((C, B), jnp.float32),
        ],
        compiler_params=pltpu.CompilerParams(
            dimension_semantics=("arbitrary",),
        ),
    )(xs, availabilities, gT, W0, W1, W2, bs, Wc, bc2)
    return out.T


# speedup vs baseline: 2.6719x; 1.3625x over previous
import jax
import jax.numpy as jnp
from jax.experimental import pallas as pl
from jax.experimental.pallas import tpu as pltpu

MU = 0.7
SIGMA = 0.2
BE = 256


def _fused(xs_ref, avail_ref, g_ref, w0_ref, w1_ref, w2_ref, bs_ref,
           wc_ref, bc_ref, out_ref, xw_s, idx_s, emb_s, wc_s, wc_sem):
    i = pl.program_id(0)
    nb = pl.num_programs(0)

    @pl.when(i == 0)
    def _init():
        pltpu.make_async_copy(wc_ref, wc_s, wc_sem).start()
        xw_s[...] = jnp.exp(-0.5 * ((xs_ref[...] - MU) / SIGMA) ** 2)
        avail = avail_ref[...]
        logsel = jnp.log(avail / jnp.sum(avail, axis=-1, keepdims=True))
        s0 = g_ref[0] + logsel[:, 0][:, None]
        s1 = g_ref[1] + logsel[:, 1][:, None]
        s2 = g_ref[2] + logsel[:, 2][:, None]
        idx_s[...] = jnp.where(
            s0 >= s1,
            jnp.where(s0 >= s2, 0, 2),
            jnp.where(s1 >= s2, 1, 2),
        ).astype(jnp.int32).T

    sl = pl.ds(i * BE, BE)
    dn = (((1,), (1,)), ((), ()))
    d0 = jax.lax.dot_general(w0_ref[...], xw_s[0], dn,
                             preferred_element_type=jnp.float32)
    d1 = jax.lax.dot_general(w1_ref[...], xw_s[1], dn,
                             preferred_element_type=jnp.float32)
    d2 = jax.lax.dot_general(w2_ref[...], xw_s[2], dn,
                             preferred_element_type=jnp.float32)
    d0 = jnp.maximum(d0 + bs_ref[0, sl][:, None], 0.0)
    d1 = jnp.maximum(d1 + bs_ref[1, sl][:, None], 0.0)
    d2 = jnp.maximum(d2 + bs_ref[2, sl][:, None], 0.0)
    idx = idx_s[sl, :]
    emb_s[sl, :] = jnp.where(idx == 0, d0, jnp.where(idx == 1, d1, d2))

    @pl.when(i == nb - 1)
    def _final():
        pltpu.make_async_copy(wc_ref, wc_s, wc_sem).wait()
        l = jax.lax.dot_general(wc_s[...], emb_s[...],
                                (((1,), (0,)), ((), ())),
                                preferred_element_type=jnp.float32)
        l = l + bc_ref[...]
        l = l - jnp.max(l, axis=0, keepdims=True)
        p = jnp.exp(l)
        out_ref[...] = p / jnp.sum(p, axis=0, keepdims=True)


def kernel(face, audio, text, availabilities, W0, b0, W1, b1, W2, b2, Wc, bc):
    B, D = face.shape
    E = W0.shape[0]
    C = Wc.shape[0]
    M = availabilities.shape[1]
    xs = jnp.stack([face, audio, text])
    bs = jnp.stack([b0, b1, b2])
    g = jax.random.gumbel(jax.random.key(42), (E, B, M), jnp.float32)
    gT = g.transpose(2, 1, 0)
    bc2 = bc.reshape(C, 1)
    nb = E // BE

    out = pl.pallas_call(
        _fused,
        grid=(nb,),
        in_specs=[
            pl.BlockSpec((M, B, D), lambda i: (0, 0, 0)),
            pl.BlockSpec((B, M), lambda i: (0, 0)),
            pl.BlockSpec((M, B, E), lambda i: (0, 0, 0)),
            pl.BlockSpec((BE, D), lambda i: (i, 0)),
            pl.BlockSpec((BE, D), lambda i: (i, 0)),
            pl.BlockSpec((BE, D), lambda i: (i, 0)),
            pl.BlockSpec((M, E), lambda i: (0, 0)),
            pl.BlockSpec(memory_space=pl.ANY),
            pl.BlockSpec((C, 1), lambda i: (0, 0)),
        ],
        out_specs=pl.BlockSpec((C, B), lambda i: (0, 0)),
        out_shape=jax.ShapeDtypeStruct((C, B), jnp.float32),
        scratch_shapes=[
            pltpu.VMEM((M, B, D), jnp.float32),
            pltpu.VMEM((E, B), jnp.int32),
            pltpu.VMEM((E, B), jnp.float32),
            pltpu.VMEM((C, E), jnp.float32),
            pltpu.SemaphoreType.DMA,
        ],
        compiler_params=pltpu.CompilerParams(
            dimension_semantics=("arbitrary",),
        ),
    )(xs, availabilities, gT, W0, W1, W2, bs, Wc, bc2)
    return out.T
